# baseline (device time: 9113 ns/iter reference)
import jax
import jax.numpy as jnp
from jax import lax
from jax.experimental import pallas as pl
from jax.experimental.pallas import tpu as pltpu

N_DEV = 4
EPS = 1e-5


def kernel(x, gamma, beta):
    m, n_per = x.shape
    n_global = n_per * N_DEV
    assert m % 128 == 0
    mr = m // 128

    import os
    _scopes = os.environ.get("KERNEL_SCOPES", "0") == "1"

    class _noscope:
        def __init__(self, name):
            self._cm = jax.named_scope(name) if _scopes else None

        def __enter__(self):
            if self._cm:
                self._cm.__enter__()

        def __exit__(self, *a):
            if self._cm:
                self._cm.__exit__(*a)

    def body(x_ref, gb_ref, out_ref,
             mystats_ref, comm_ref, gxf_ref,
             send_sems, recv_sems):
        my = lax.axis_index("i")

        with _noscope("phase_signal"):
            barrier = pltpu.get_barrier_semaphore()
            for k in range(1, N_DEV):
                peer = lax.rem(my + k, N_DEV)
                pl.semaphore_signal(
                    barrier, inc=1,
                    device_id=(peer,), device_id_type=pl.DeviceIdType.MESH,
                )

        xf = x_ref[...]
        ri = lax.broadcasted_iota(jnp.int32, (m, 128), 0)
        ci = lax.broadcasted_iota(jnp.int32, (m, 128), 1)
        mask = (ri % 128 == ci).astype(jnp.float32)
        rt = (lax.broadcasted_iota(jnp.int32, (mr, m), 1) // 128
              == lax.broadcasted_iota(jnp.int32, (mr, m), 0)
              ).astype(jnp.float32)
        r_ = (lax.broadcasted_iota(jnp.int32, (m, mr), 0) // 128
              == lax.broadcasted_iota(jnp.int32, (m, mr), 1)
              ).astype(jnp.float32)

        with _noscope("phase_stats"):
            ones_col = jnp.ones((n_per, 1), jnp.float32)
            s_col = jnp.dot(xf, ones_col,
                            preferred_element_type=jnp.float32)
            ss_col = jnp.dot(xf * xf, ones_col,
                             preferred_element_type=jnp.float32)
            mystats_ref[0:mr] = jnp.dot(
                rt, s_col * mask, preferred_element_type=jnp.float32)
            mystats_ref[mr:2 * mr] = jnp.dot(
                rt, ss_col * mask, preferred_element_type=jnp.float32)

        with _noscope("phase_barrier_wait"):
            pl.semaphore_wait(barrier, N_DEV - 1)

        rdmas = []
        with _noscope("phase_rdma_start"):
            for k in range(1, N_DEV):
                peer = lax.rem(my + k, N_DEV)
                rdma = pltpu.make_async_remote_copy(
                    src_ref=mystats_ref,
                    dst_ref=comm_ref.at[k - 1],
                    send_sem=send_sems.at[k - 1],
                    recv_sem=recv_sems.at[k - 1],
                    device_id=(peer,),
                    device_id_type=pl.DeviceIdType.MESH,
                )
                rdma.start()
                rdmas.append(rdma)

        with _noscope("phase_gxf"):
            g_row = gb_ref[0:n_per][None, :]
            b_row = gb_ref[n_per:2 * n_per][None, :]
            gxf_ref[...] = (g_row * xf).astype(jnp.bfloat16)

        with _noscope("phase_wait_recv"):
            for rdma in rdmas:
                rdma.wait_recv()

        with _noscope("phase_normalize"):
            total = mystats_ref[...]
            for k in range(N_DEV - 1):
                total = total + comm_ref[k]

            def unpack(t):
                big = jnp.dot(r_, t, preferred_element_type=jnp.float32)
                return jnp.sum(big * mask, axis=1, keepdims=True)

            mean = unpack(total[0:mr]) * (1.0 / n_global)
            ex2 = unpack(total[mr:2 * mr]) * (1.0 / n_global)
            var = ex2 - mean * mean
            inv = lax.rsqrt(var + EPS)
            out_ref[...] = (gxf_ref[...] * inv - g_row * (mean * inv)
                            + b_row).astype(out_ref.dtype)

        with _noscope("phase_wait_send"):
            for rdma in rdmas:
                rdma.wait_send()

    return pl.pallas_call(
        body,
        out_shape=jax.ShapeDtypeStruct((m, n_per), jnp.bfloat16),
        in_specs=[pl.BlockSpec(memory_space=pltpu.VMEM)] * 2,
        out_specs=pl.BlockSpec(memory_space=pltpu.VMEM),
        scratch_shapes=[
            pltpu.VMEM((2 * mr, 128), jnp.float32),
            pltpu.VMEM((N_DEV - 1, 2 * mr, 128), jnp.float32),
            pltpu.VMEM((m, n_per), jnp.bfloat16),
            pltpu.SemaphoreType.DMA((N_DEV - 1,)),
            pltpu.SemaphoreType.DMA((N_DEV - 1,)),
        ],
        compiler_params=pltpu.CompilerParams(collective_id=0),
    )(x, jnp.concatenate([gamma, beta]))


# device time: 8645 ns/iter; 1.0541x vs baseline; 1.0541x over previous
import jax
import jax.numpy as jnp
from jax import lax
from jax.experimental import pallas as pl
from jax.experimental.pallas import tpu as pltpu

N_DEV = 4
EPS = 1e-5


def kernel(x, gamma, beta):
    m, n_per = x.shape
    n_global = n_per * N_DEV
    assert m % 128 == 0
    mr = m // 128

    import os
    _scopes = os.environ.get("KERNEL_SCOPES", "0") == "1"

    class _noscope:
        def __init__(self, name):
            self._cm = jax.named_scope(name) if _scopes else None

        def __enter__(self):
            if self._cm:
                self._cm.__enter__()

        def __exit__(self, *a):
            if self._cm:
                self._cm.__exit__(*a)

    def body(x_ref, gb_ref, out_ref,
             mystats_ref, comm_ref, gxf_ref,
             send_sems, recv_sems):
        my = lax.axis_index("i")

        with _noscope("phase_signal"):
            barrier = pltpu.get_barrier_semaphore()
            for k in range(1, N_DEV):
                peer = lax.rem(my + k, N_DEV)
                pl.semaphore_signal(
                    barrier, inc=1,
                    device_id=(peer,), device_id_type=pl.DeviceIdType.MESH,
                )

        xf = x_ref[...]
        x3 = xf.reshape(mr, 128, n_per)
        with _noscope("phase_stats"):
            mystats_ref[0:mr] = jnp.sum(x3, axis=2)
            mystats_ref[mr:2 * mr] = jnp.sum(x3 * x3, axis=2)

        with _noscope("phase_barrier_wait"):
            pl.semaphore_wait(barrier, N_DEV - 1)

        rdmas = []
        with _noscope("phase_rdma_start"):
            for k in range(1, N_DEV):
                peer = lax.rem(my + k, N_DEV)
                rdma = pltpu.make_async_remote_copy(
                    src_ref=mystats_ref,
                    dst_ref=comm_ref.at[k - 1],
                    send_sem=send_sems.at[k - 1],
                    recv_sem=recv_sems.at[k - 1],
                    device_id=(peer,),
                    device_id_type=pl.DeviceIdType.MESH,
                )
                rdma.start()
                rdmas.append(rdma)

        with _noscope("phase_gxf"):
            g_row = gb_ref[0:n_per][None, :]
            b_row = gb_ref[n_per:2 * n_per][None, :]
            gxf_ref[...] = (g_row * xf).astype(jnp.bfloat16)

        with _noscope("phase_wait_recv"):
            for rdma in rdmas:
                rdma.wait_recv()

        with _noscope("phase_normalize"):
            total = mystats_ref[...]
            for k in range(N_DEV - 1):
                total = total + comm_ref[k]

            mean = total[0:mr, :, None] * (1.0 / n_global)
            ex2 = total[mr:2 * mr, :, None] * (1.0 / n_global)
            var = ex2 - mean * mean
            inv = lax.rsqrt(var + EPS)
            gxf3 = gxf_ref[...].reshape(mr, 128, n_per)
            g3 = g_row[None]
            b3 = b_row[None]
            out3 = gxf3 * inv - g3 * (mean * inv) + b3
            out_ref[...] = out3.reshape(m, n_per).astype(out_ref.dtype)

        with _noscope("phase_wait_send"):
            for rdma in rdmas:
                rdma.wait_send()

    return pl.pallas_call(
        body,
        out_shape=jax.ShapeDtypeStruct((m, n_per), jnp.bfloat16),
        in_specs=[pl.BlockSpec(memory_space=pltpu.VMEM)] * 2,
        out_specs=pl.BlockSpec(memory_space=pltpu.VMEM),
        scratch_shapes=[
            pltpu.VMEM((2 * mr, 128), jnp.float32),
            pltpu.VMEM((N_DEV - 1, 2 * mr, 128), jnp.float32),
            pltpu.VMEM((m, n_per), jnp.bfloat16),
            pltpu.SemaphoreType.DMA((N_DEV - 1,)),
            pltpu.SemaphoreType.DMA((N_DEV - 1,)),
        ],
        compiler_params=pltpu.CompilerParams(collective_id=0),
    )(x, jnp.concatenate([gamma, beta]))
